# Initial kernel scaffold; baseline (speedup 1.0000x reference)
import jax
import jax.numpy as jnp
from jax.experimental import pallas as pl


def kernel(feat, xyz, W_feat, b_feat, W_byp, g_byp, be_byp, W_ag, b_ag, g1, b1, g2, b2):
    raise NotImplementedError("write your pallas kernel here")


# speedup vs baseline: 2.3858x; 2.3858x over previous
import jax
import jax.numpy as jnp
from jax.experimental import pallas as pl
from jax.experimental.pallas import tpu as pltpu

RADIUS, DECAY_RADIUS = 0.15, 0.3
EPS_BN = 1e-5
B, N = 4, 1024
C_IN, C_OUT, C_BYP = 64, 64, 32
C6 = 6 * C_BYP
TI = 256
RL = 512
BN_CNT = B * N


def _k_linear(feat_ref, wf_ref, bf_ref, wb_ref, self_ref, mut_ref, stats_ref):
    f = feat_ref[...]
    self_ref[...] = jnp.dot(f, wf_ref[...], preferred_element_type=jnp.float32) + bf_ref[...]
    m = jnp.dot(f, wb_ref[...], preferred_element_type=jnp.float32)
    mut_ref[...] = m

    @pl.when(pl.program_id(0) == 0)
    def _():
        stats_ref[...] = jnp.zeros_like(stats_ref)

    s1 = jnp.sum(m, axis=0, keepdims=True)
    s2 = jnp.sum(m * m, axis=0, keepdims=True)
    stats_ref[0:1, 0:C6] += s1
    stats_ref[1:2, 0:C6] += s2


def _k_aggregate(mut_ref, xyz_ref, xyzt_ref, a_ref, b_ref, ag_ref, stats_ref):
    i = pl.program_id(1)
    g = jnp.maximum(mut_ref[0] * a_ref[...] + b_ref[...], 0.0)

    xi = xyz_ref[0, pl.ds(i * TI, TI), :]
    xj = xyzt_ref[0]
    dx = xj[0:1, :] - xi[:, 0:1]
    dy = xj[1:2, :] - xi[:, 1:2]
    dz = xj[2:3, :] - xi[:, 2:3]
    dist2 = dx * dx + dy * dy + dz * dz
    r2 = RADIUS * RADIUS
    dr2 = DECAY_RADIUS * DECAY_RADIUS
    decay = (dr2 - dist2) * (1.0 / (dr2 - r2))
    w = jnp.where(dist2 <= r2, 1.0, decay)
    w = jnp.where((dist2 < dr2) & (dist2 > 0.0), w, 0.0)
    u = w / jnp.maximum(dist2, 1e-12)
    norm = jnp.sum(w, axis=1, keepdims=True)

    acc = jnp.zeros((TI, C_BYP), dtype=jnp.float32)
    for k, d in enumerate((dx, -dx, dy, -dy, dz, -dz)):
        comp = jnp.maximum(d, 0.0)
        A = u * comp * comp
        acc += jnp.dot(A, g[:, k * C_BYP:(k + 1) * C_BYP],
                       preferred_element_type=jnp.float32)

    ag = acc / jnp.maximum(norm, 1e-8)
    ag_ref[0] = ag

    @pl.when(jnp.logical_and(pl.program_id(0) == 0, i == 0))
    def _():
        stats_ref[...] = jnp.zeros_like(stats_ref)

    stats_ref[0:1, 0:C_BYP] += jnp.sum(ag, axis=0, keepdims=True)
    stats_ref[1:2, 0:C_BYP] += jnp.sum(ag * ag, axis=0, keepdims=True)


def _k_head(ag_ref, a1_ref, b1_ref, wag_ref, bag_ref, self_ref, pre_ref, stats_ref):
    agn = jnp.maximum(ag_ref[...] * a1_ref[...] + b1_ref[...], 0.0)
    pre = (jnp.dot(agn, wag_ref[...], preferred_element_type=jnp.float32)
           + bag_ref[...] + self_ref[...])
    pre_ref[...] = pre

    @pl.when(pl.program_id(0) == 0)
    def _():
        stats_ref[...] = jnp.zeros_like(stats_ref)

    stats_ref[0:1, 0:C_OUT] += jnp.sum(pre, axis=0, keepdims=True)
    stats_ref[1:2, 0:C_OUT] += jnp.sum(pre * pre, axis=0, keepdims=True)


def _k_bnrelu(x_ref, a_ref, b_ref, o_ref):
    o_ref[...] = jnp.maximum(x_ref[...] * a_ref[...] + b_ref[...], 0.0)


def _bn_fold(s1, s2, gamma, beta):
    mean = s1 / BN_CNT
    var = s2 / BN_CNT - mean * mean
    a = gamma / jnp.sqrt(var + EPS_BN)
    return a, beta - mean * a


def kernel(feat, xyz, W_feat, b_feat, W_byp, g_byp, be_byp, W_ag, b_ag, g1, b1, g2, b2):
    f2 = feat.reshape(B * N, C_IN)

    self_feat, mut_pre, st0 = pl.pallas_call(
        _k_linear,
        grid=(B * N // RL,),
        in_specs=[
            pl.BlockSpec((RL, C_IN), lambda r: (r, 0)),
            pl.BlockSpec((C_IN, C_OUT), lambda r: (0, 0)),
            pl.BlockSpec((1, C_OUT), lambda r: (0, 0)),
            pl.BlockSpec((C_IN, C6), lambda r: (0, 0)),
        ],
        out_specs=[
            pl.BlockSpec((RL, C_OUT), lambda r: (r, 0)),
            pl.BlockSpec((RL, C6), lambda r: (r, 0)),
            pl.BlockSpec((8, 256), lambda r: (0, 0)),
        ],
        out_shape=[
            jax.ShapeDtypeStruct((B * N, C_OUT), jnp.float32),
            jax.ShapeDtypeStruct((B * N, C6), jnp.float32),
            jax.ShapeDtypeStruct((8, 256), jnp.float32),
        ],
    )(f2, W_feat.T, b_feat.reshape(1, C_OUT), W_byp.T)

    a_byp, sh_byp = _bn_fold(st0[0, 0:C6], st0[1, 0:C6], g_byp, be_byp)

    ag, st1 = pl.pallas_call(
        _k_aggregate,
        grid=(B, N // TI),
        in_specs=[
            pl.BlockSpec((1, N, C6), lambda b, i: (b, 0, 0)),
            pl.BlockSpec((1, N, 3), lambda b, i: (b, 0, 0)),
            pl.BlockSpec((1, 3, N), lambda b, i: (b, 0, 0)),
            pl.BlockSpec((1, C6), lambda b, i: (0, 0)),
            pl.BlockSpec((1, C6), lambda b, i: (0, 0)),
        ],
        out_specs=[
            pl.BlockSpec((1, TI, C_BYP), lambda b, i: (b, i, 0)),
            pl.BlockSpec((8, 128), lambda b, i: (0, 0)),
        ],
        out_shape=[
            jax.ShapeDtypeStruct((B, N, C_BYP), jnp.float32),
            jax.ShapeDtypeStruct((8, 128), jnp.float32),
        ],
    )(mut_pre.reshape(B, N, C6), xyz, jnp.transpose(xyz, (0, 2, 1)),
      a_byp.reshape(1, C6), sh_byp.reshape(1, C6))

    a1, sh1 = _bn_fold(st1[0, 0:C_BYP], st1[1, 0:C_BYP], g1, b1)

    pre, st2 = pl.pallas_call(
        _k_head,
        grid=(B * N // RL,),
        in_specs=[
            pl.BlockSpec((RL, C_BYP), lambda r: (r, 0)),
            pl.BlockSpec((1, C_BYP), lambda r: (0, 0)),
            pl.BlockSpec((1, C_BYP), lambda r: (0, 0)),
            pl.BlockSpec((C_BYP, C_OUT), lambda r: (0, 0)),
            pl.BlockSpec((1, C_OUT), lambda r: (0, 0)),
            pl.BlockSpec((RL, C_OUT), lambda r: (r, 0)),
        ],
        out_specs=[
            pl.BlockSpec((RL, C_OUT), lambda r: (r, 0)),
            pl.BlockSpec((8, 128), lambda r: (0, 0)),
        ],
        out_shape=[
            jax.ShapeDtypeStruct((B * N, C_OUT), jnp.float32),
            jax.ShapeDtypeStruct((8, 128), jnp.float32),
        ],
    )(ag.reshape(B * N, C_BYP), a1.reshape(1, C_BYP), sh1.reshape(1, C_BYP),
      W_ag.T, b_ag.reshape(1, C_OUT), self_feat)

    a2, sh2 = _bn_fold(st2[0, 0:C_OUT], st2[1, 0:C_OUT], g2, b2)

    out = pl.pallas_call(
        _k_bnrelu,
        grid=(B * N // RL,),
        in_specs=[
            pl.BlockSpec((RL, C_OUT), lambda r: (r, 0)),
            pl.BlockSpec((1, C_OUT), lambda r: (0, 0)),
            pl.BlockSpec((1, C_OUT), lambda r: (0, 0)),
        ],
        out_specs=pl.BlockSpec((RL, C_OUT), lambda r: (r, 0)),
        out_shape=jax.ShapeDtypeStruct((B * N, C_OUT), jnp.float32),
    )(pre, a2.reshape(1, C_OUT), sh2.reshape(1, C_OUT))

    return out.reshape(B, N, C_OUT)


# speedup vs baseline: 3.8804x; 1.6265x over previous
import jax
import jax.numpy as jnp
from jax import lax
from jax.experimental import pallas as pl
from jax.experimental.pallas import tpu as pltpu

RADIUS, DECAY_RADIUS = 0.15, 0.3
EPS_BN = 1e-5
B, N = 4, 1024
C_IN, C_OUT, C_BYP = 64, 64, 32
C6 = 6 * C_BYP
TI = 256
BN_CNT = B * N
_R2 = RADIUS * RADIUS
_DR2 = DECAY_RADIUS * DECAY_RADIUS
_C1 = 1.0 / (_DR2 - _R2)
_C0 = _DR2 * _C1


def _bn_fold(x, gamma, beta):
    s1 = jnp.sum(x, axis=0, keepdims=True)
    s2 = jnp.sum(x * x, axis=0, keepdims=True)
    mean = s1 * (1.0 / BN_CNT)
    var = s2 * (1.0 / BN_CNT) - mean * mean
    a = gamma * lax.rsqrt(var + EPS_BN)
    return a, beta - mean * a


def _k_all(feat_ref, xyz_ref, xyzt_ref, wf_ref, bf_ref, wb_ref, gb_ref, beb_ref,
           wag_ref, bag_ref, g1_ref, b1_ref, g2_ref, b2_ref, out_ref, ag_scr):
    feat = feat_ref[...]
    self_feat = jnp.dot(feat, wf_ref[...], preferred_element_type=jnp.float32) + bf_ref[...]
    mut = jnp.dot(feat, wb_ref[...], preferred_element_type=jnp.float32)

    a_b, sh_b = _bn_fold(mut, gb_ref[...], beb_ref[...])

    ones_n = jnp.ones((N, C_BYP), dtype=jnp.float32)
    for b in range(B):
        g = jnp.maximum(mut[b * N:(b + 1) * N, :] * a_b + sh_b, 0.0)
        xj = xyzt_ref[b]

        def body(i, _):
            xi = xyz_ref[b, pl.ds(i * TI, TI), :]
            dx = xj[0:1, :] - xi[:, 0:1]
            dy = xj[1:2, :] - xi[:, 1:2]
            dz = xj[2:3, :] - xi[:, 2:3]
            sqx = dx * dx
            sqy = dy * dy
            sqz = dz * dz
            dist2 = sqx + sqy + sqz
            rcp = 1.0 / jnp.maximum(dist2, 1e-12)
            w = jnp.clip(_C0 - dist2 * _C1, 0.0, 1.0)
            w = jnp.where(dist2 > 0.0, w, 0.0)
            u = w * rcp

            acc = jnp.zeros((TI, C_BYP), dtype=jnp.float32)
            for ax, (d, sq) in enumerate(((dx, sqx), (dy, sqy), (dz, sqz))):
                q = u * sq
                ap = jnp.where(d > 0.0, q, 0.0)
                am = q - ap
                acc += jnp.dot(ap, g[:, (2 * ax) * C_BYP:(2 * ax + 1) * C_BYP],
                               preferred_element_type=jnp.float32)
                acc += jnp.dot(am, g[:, (2 * ax + 1) * C_BYP:(2 * ax + 2) * C_BYP],
                               preferred_element_type=jnp.float32)
            norm = jnp.dot(w, ones_n, preferred_element_type=jnp.float32)
            ag_scr[pl.ds(b * N + i * TI, TI), :] = acc / jnp.maximum(norm, 1e-8)
            return _

        lax.fori_loop(0, N // TI, body, 0, unroll=True)

    ag = ag_scr[...]
    a1, sh1 = _bn_fold(ag, g1_ref[...], b1_ref[...])
    agn = jnp.maximum(ag * a1 + sh1, 0.0)
    pre = (jnp.dot(agn, wag_ref[...], preferred_element_type=jnp.float32)
           + bag_ref[...] + self_feat)
    a2, sh2 = _bn_fold(pre, g2_ref[...], b2_ref[...])
    out_ref[...] = jnp.maximum(pre * a2 + sh2, 0.0)


def kernel(feat, xyz, W_feat, b_feat, W_byp, g_byp, be_byp, W_ag, b_ag, g1, b1, g2, b2):
    out = pl.pallas_call(
        _k_all,
        out_shape=jax.ShapeDtypeStruct((B * N, C_OUT), jnp.float32),
        scratch_shapes=[pltpu.VMEM((B * N, C_BYP), jnp.float32)],
    )(feat.reshape(B * N, C_IN), xyz, jnp.transpose(xyz, (0, 2, 1)),
      W_feat.T, b_feat.reshape(1, C_OUT), W_byp.T,
      g_byp.reshape(1, C6), be_byp.reshape(1, C6),
      W_ag.T, b_ag.reshape(1, C_OUT),
      g1.reshape(1, C_BYP), b1.reshape(1, C_BYP),
      g2.reshape(1, C_OUT), b2.reshape(1, C_OUT))
    return out.reshape(B, N, C_OUT)
